# baseline (speedup 1.0000x reference)
import functools

import jax
import jax.numpy as jnp
from jax import lax
from jax.experimental import pallas as pl
from jax.experimental.pallas import tpu as pltpu
from jax.experimental.pallas import tpu_sc as plsc

D_MODEL = 1024
SUB = 8
LANES = 128
SEQ = 4096
NUM_CORES = 2
NUM_SUBCORES = 16
NUM_WORKERS = NUM_CORES * NUM_SUBCORES
CHUNK = 32


def _make_sc_gather(n_rows):
    rpw = n_rows // NUM_WORKERS
    n_chunks = rpw // CHUNK
    assert n_chunks % 2 == 0 and n_chunks >= 2

    @functools.partial(
        pl.kernel,
        out_type=jax.ShapeDtypeStruct((n_rows, SUB, LANES), jnp.float32),
        mesh=plsc.VectorSubcoreMesh(core_axis_name="c", subcore_axis_name="s"),
        scratch_types=[
            pltpu.VMEM((rpw,), jnp.int32),
            pltpu.VMEM((CHUNK, SUB, LANES), jnp.float32),
            pltpu.VMEM((CHUNK, SUB, LANES), jnp.float32),
            pltpu.SemaphoreType.DMA,
            pltpu.SemaphoreType.DMA,
            pltpu.SemaphoreType.DMA,
            pltpu.SemaphoreType.DMA,
        ],
    )
    def _sc_gather(pe_hbm, idx_hbm, out_hbm, idx_v, buf0, buf1, sg0, sg1, sw0, sw1):
        wid = lax.axis_index("s") * NUM_CORES + lax.axis_index("c")
        base = wid * rpw
        pltpu.sync_copy(idx_hbm.at[pl.ds(base, rpw)], idx_v)
        bufs, sgs, sws = (buf0, buf1), (sg0, sg1), (sw0, sw1)
        gathers = [None, None]
        writes = [None, None]
        for c in range(n_chunks):
            p = c & 1
            if writes[p] is not None:
                writes[p].wait()
            gathers[p] = pltpu.async_copy(
                pe_hbm.at[idx_v.at[pl.ds(c * CHUNK, CHUNK)]], bufs[p], sgs[p])
            if p == 1:
                for cc in (c - 1, c):
                    pp = cc & 1
                    gathers[pp].wait()
                    writes[pp] = pltpu.async_copy(
                        bufs[pp], out_hbm.at[pl.ds(base + cc * CHUNK, CHUNK)], sws[pp])
        writes[0].wait()
        writes[1].wait()

    return _sc_gather


_sc_gather_half = _make_sc_gather(SEQ // 2)


def _tc_add_body(x_ref, g_ref, o_ref):
    for j in range(SUB):
        sl = slice(j * LANES, (j + 1) * LANES)
        o_ref[:, :, sl] = x_ref[:, :, sl] + g_ref[:, j, :][None]


def _tc_add_alias_body(prev_ref, x_ref, g_ref, o_ref):
    del prev_ref
    _tc_add_body(x_ref, g_ref, o_ref)


BS = 512
HALF = SEQ // 2
HALF_BLOCKS = HALF // BS


def _tc_add_half(x, gh, prev, half):
    b, s, d = x.shape
    off = half * HALF_BLOCKS
    if prev is None:
        return pl.pallas_call(
            _tc_add_body,
            grid=(HALF_BLOCKS,),
            in_specs=[
                pl.BlockSpec((b, BS, d), lambda i: (0, i + off, 0)),
                pl.BlockSpec((BS, SUB, LANES), lambda i: (i, 0, 0)),
            ],
            out_specs=pl.BlockSpec((b, BS, d), lambda i: (0, i + off, 0)),
            out_shape=jax.ShapeDtypeStruct((b, s, d), x.dtype),
        )(x, gh)
    return pl.pallas_call(
        _tc_add_alias_body,
        grid=(HALF_BLOCKS,),
        in_specs=[
            pl.BlockSpec(memory_space=pl.ANY),
            pl.BlockSpec((b, BS, d), lambda i: (0, i + off, 0)),
            pl.BlockSpec((BS, SUB, LANES), lambda i: (i, 0, 0)),
        ],
        out_specs=pl.BlockSpec((b, BS, d), lambda i: (0, i + off, 0)),
        out_shape=jax.ShapeDtypeStruct((b, s, d), x.dtype),
        input_output_aliases={0: 0},
    )(prev, x, gh)


def kernel(x, time_indices, pe):
    idx = time_indices.astype(jnp.int32)
    pe3 = pe.reshape(pe.shape[0], SUB, LANES)
    g0 = _sc_gather_half(pe3, idx[:HALF])
    g1 = _sc_gather_half(pe3, idx[HALF:])
    out = _tc_add_half(x, g0, None, 0)
    out = _tc_add_half(x, g1, out, 1)
    return out


# speedup vs baseline: 1.0255x; 1.0255x over previous
import functools

import jax
import jax.numpy as jnp
from jax import lax
from jax.experimental import pallas as pl
from jax.experimental.pallas import tpu as pltpu
from jax.experimental.pallas import tpu_sc as plsc

D_MODEL = 1024
SUB = 8
LANES = 128
SEQ = 4096
NUM_CORES = 2
NUM_SUBCORES = 16
NUM_WORKERS = NUM_CORES * NUM_SUBCORES
CHUNK = 32


def _make_sc_gather(n_rows):
    rpw = n_rows // NUM_WORKERS
    n_chunks = rpw // CHUNK
    assert n_chunks % 2 == 0 and n_chunks >= 2

    @functools.partial(
        pl.kernel,
        out_type=jax.ShapeDtypeStruct((n_rows, SUB, LANES), jnp.float32),
        mesh=plsc.VectorSubcoreMesh(core_axis_name="c", subcore_axis_name="s"),
        scratch_types=[
            pltpu.VMEM((rpw,), jnp.int32),
            pltpu.VMEM((CHUNK, SUB, LANES), jnp.float32),
            pltpu.VMEM((CHUNK, SUB, LANES), jnp.float32),
            pltpu.SemaphoreType.DMA,
            pltpu.SemaphoreType.DMA,
            pltpu.SemaphoreType.DMA,
            pltpu.SemaphoreType.DMA,
        ],
    )
    def _sc_gather(pe_hbm, idx_hbm, out_hbm, idx_v, buf0, buf1, sg0, sg1, sw0, sw1):
        wid = lax.axis_index("s") * NUM_CORES + lax.axis_index("c")
        base = wid * rpw
        pltpu.sync_copy(idx_hbm.at[pl.ds(base, rpw)], idx_v)
        bufs, sgs, sws = (buf0, buf1), (sg0, sg1), (sw0, sw1)
        gathers = [None, None]
        writes = [None, None]
        for c in range(n_chunks):
            p = c & 1
            if writes[p] is not None:
                writes[p].wait()
            gathers[p] = pltpu.async_copy(
                pe_hbm.at[idx_v.at[pl.ds(c * CHUNK, CHUNK)]], bufs[p], sgs[p])
            if p == 1:
                for cc in (c - 1, c):
                    pp = cc & 1
                    gathers[pp].wait()
                    writes[pp] = pltpu.async_copy(
                        bufs[pp], out_hbm.at[pl.ds(base + cc * CHUNK, CHUNK)], sws[pp])
        writes[0].wait()
        writes[1].wait()

    return _sc_gather


_sc_gather_full = _make_sc_gather(SEQ)

BS = 512


def _tc_add_body(x_ref, g_ref, o_ref):
    for j in range(SUB):
        sl = slice(j * LANES, (j + 1) * LANES)
        o_ref[:, :, sl] = x_ref[:, :, sl] + g_ref[:, j, :][None]


def kernel(x, time_indices, pe):
    idx = time_indices.astype(jnp.int32)
    pe3 = pe.reshape(pe.shape[0], SUB, LANES)
    g = _sc_gather_full(pe3, idx)
    b, s, d = x.shape
    return pl.pallas_call(
        _tc_add_body,
        grid=(s // BS,),
        in_specs=[
            pl.BlockSpec((b, BS, d), lambda i: (0, i, 0)),
            pl.BlockSpec((BS, SUB, LANES), lambda i: (i, 0, 0)),
        ],
        out_specs=pl.BlockSpec((b, BS, d), lambda i: (0, i, 0)),
        out_shape=jax.ShapeDtypeStruct((b, s, d), x.dtype),
    )(x, g)
